# Initial kernel scaffold; baseline (speedup 1.0000x reference)
import jax
import jax.numpy as jnp
from jax.experimental import pallas as pl


def kernel(x, z, w1, b1, w2, b2):
    raise NotImplementedError("write your pallas kernel here")


# speedup vs baseline: 1.3123x; 1.3123x over previous
import functools

import jax
import jax.numpy as jnp
from jax.experimental import pallas as pl
from jax.experimental.pallas import tpu as pltpu


def _se_fused_kernel(x_ref, z_ref, w1_ref, b1_ref, w2_ref, b2_ref, o_ref,
                     *, inv_hw):
    pooled = jnp.sum(x_ref[...].astype(jnp.float32), axis=1,
                     keepdims=True) * inv_hw
    h = jnp.dot(w1_ref[...], pooled,
                preferred_element_type=jnp.float32) + b1_ref[...]
    h = h * jax.nn.sigmoid(h)
    y = jnp.dot(w2_ref[...], h,
                preferred_element_type=jnp.float32) + b2_ref[...]
    s = jax.nn.sigmoid(y)
    o_ref[...] = (s * z_ref[...].astype(jnp.float32)).astype(o_ref.dtype)


def kernel(x, z, w1, b1, w2, b2):
    B, inp, H, W = x.shape
    Bz, oup, Hz, Wz = z.shape
    assert B == Bz
    sq = w1.shape[0]
    HW, HWz = H * W, Hz * Wz

    x2 = x.reshape(B * inp, HW)
    z2 = z.reshape(B * oup, HWz)
    w1f = w1.astype(jnp.float32)
    w2f = w2.astype(jnp.float32)
    b1c = b1.astype(jnp.float32).reshape(sq, 1)
    b2c = b2.astype(jnp.float32).reshape(oup, 1)

    out2 = pl.pallas_call(
        functools.partial(_se_fused_kernel, inv_hw=float(1.0 / HW)),
        out_shape=jax.ShapeDtypeStruct((B * oup, HWz), z.dtype),
        grid=(B,),
        in_specs=[
            pl.BlockSpec((inp, HW), lambda b: (b, 0)),
            pl.BlockSpec((oup, HWz), lambda b: (b, 0)),
            pl.BlockSpec((sq, inp), lambda b: (0, 0)),
            pl.BlockSpec((sq, 1), lambda b: (0, 0)),
            pl.BlockSpec((oup, sq), lambda b: (0, 0)),
            pl.BlockSpec((oup, 1), lambda b: (0, 0)),
        ],
        out_specs=pl.BlockSpec((oup, HWz), lambda b: (b, 0)),
        compiler_params=pltpu.CompilerParams(
            dimension_semantics=("parallel",),
            vmem_limit_bytes=48 * 1024 * 1024),
    )(x2, z2, w1f, b1c, w2f, b2c)

    return out2.reshape(B, oup, Hz, Wz)


# speedup vs baseline: 2.3100x; 1.7602x over previous
import functools

import jax
import jax.numpy as jnp
from jax.experimental import pallas as pl
from jax.experimental.pallas import tpu as pltpu


def _se_fused_kernel(x_ref, z_ref, w1t_ref, b1_ref, w2t_ref, b2_ref, o_ref,
                     *, inv_hw):
    pooled = jnp.sum(x_ref[...].astype(jnp.float32), axis=(1, 2)) * inv_hw
    pooled = pooled.reshape(1, -1)
    h = jnp.dot(pooled, w1t_ref[...],
                preferred_element_type=jnp.float32) + b1_ref[...]
    h = h * jax.nn.sigmoid(h)
    y = jnp.dot(h, w2t_ref[...],
                preferred_element_type=jnp.float32) + b2_ref[...]
    s = jax.nn.sigmoid(y)
    s3 = s.reshape(-1, 1, 1)
    o_ref[...] = (s3 * z_ref[...].astype(jnp.float32)).astype(o_ref.dtype)


def kernel(x, z, w1, b1, w2, b2):
    B, inp, H, W = x.shape
    Bz, oup, Hz, Wz = z.shape
    assert B == Bz
    sq = w1.shape[0]

    x3 = x.reshape(B * inp, H, W)
    z3 = z.reshape(B * oup, Hz, Wz)
    w1t = w1.astype(jnp.float32).T
    w2t = w2.astype(jnp.float32).T
    b1r = b1.astype(jnp.float32).reshape(1, sq)
    b2r = b2.astype(jnp.float32).reshape(1, oup)

    out3 = pl.pallas_call(
        functools.partial(_se_fused_kernel, inv_hw=float(1.0 / (H * W))),
        out_shape=jax.ShapeDtypeStruct((B * oup, Hz, Wz), z.dtype),
        grid=(B,),
        in_specs=[
            pl.BlockSpec((inp, H, W), lambda b: (b, 0, 0)),
            pl.BlockSpec((oup, Hz, Wz), lambda b: (b, 0, 0)),
            pl.BlockSpec((inp, sq), lambda b: (0, 0)),
            pl.BlockSpec((1, sq), lambda b: (0, 0)),
            pl.BlockSpec((sq, oup), lambda b: (0, 0)),
            pl.BlockSpec((1, oup), lambda b: (0, 0)),
        ],
        out_specs=pl.BlockSpec((oup, Hz, Wz), lambda b: (b, 0, 0)),
        compiler_params=pltpu.CompilerParams(
            dimension_semantics=("parallel",),
            vmem_limit_bytes=56 * 1024 * 1024),
    )(x3, z3, w1t, b1r, w2t, b2r)

    return out3.reshape(B, oup, Hz, Wz)
